# Initial kernel scaffold; baseline (speedup 1.0000x reference)
import jax
import jax.numpy as jnp
from jax.experimental import pallas as pl


def kernel(x, pos_codes, struct_w, abs_emb):
    raise NotImplementedError("write your pallas kernel here")


# speedup vs baseline: 1.9775x; 1.9775x over previous
import jax
import jax.numpy as jnp
from jax.experimental import pallas as pl

_BS = 512


def _body(codes_ref, x_ref, w_ref, abs_ref, o_ref):
    acc = x_ref[0] + abs_ref[...]
    codes = codes_ref[0]
    for r in range(1, 5):
        mask = (codes == r).astype(jnp.float32)
        acc = acc + mask * w_ref[r : r + 1, :]
    o_ref[0] = acc


def kernel(x, pos_codes, struct_w, abs_emb):
    b, s, d = x.shape
    codes3 = pos_codes.astype(jnp.int32).reshape(b, s, 1)
    n_s = s // _BS
    grid = (n_s, b)
    out = pl.pallas_call(
        _body,
        grid=grid,
        in_specs=[
            pl.BlockSpec((1, _BS, 1), lambda si, bi: (bi, si, 0)),
            pl.BlockSpec((1, _BS, d), lambda si, bi: (bi, si, 0)),
            pl.BlockSpec((5, d), lambda si, bi: (0, 0)),
            pl.BlockSpec((_BS, d), lambda si, bi: (si, 0)),
        ],
        out_specs=pl.BlockSpec((1, _BS, d), lambda si, bi: (bi, si, 0)),
        out_shape=jax.ShapeDtypeStruct((b, s, d), x.dtype),
    )(codes3, x, struct_w, abs_emb)
    return out
